# baseline (speedup 1.0000x reference)
import functools

import jax
import jax.numpy as jnp
import numpy as np
from jax import lax
from jax.experimental import pallas as pl
from jax.experimental.pallas import tpu as pltpu
from jax.experimental.pallas import tpu_sc as plsc

NC = 2
NS = 16
NW = NC * NS
LANES = 16
NB = 8
H = 128

_PERM = np.empty((H,), np.int32)
for _j in range(H // 32):
  for _t in range(32):
    _PERM[_j * 32 + _t] = _j * 32 + (_t // 2 if _t % 2 == 0 else 16 + _t // 2)


def _sc_gather_rows(emb, idx_pad):
  B = idx_pad.shape[0]
  D = emb.shape[1]
  assert B % (8 * NW) == 0
  bpw = B // NW
  mesh = plsc.VectorSubcoreMesh(core_axis_name="c", subcore_axis_name="s")

  @functools.partial(
      pl.kernel,
      out_type=jax.ShapeDtypeStruct((B, D), jnp.float32),
      mesh=mesh,
      scratch_types=[
          pltpu.VMEM((bpw,), jnp.int32),
          pltpu.VMEM((bpw, D), jnp.float32),
          pltpu.SemaphoreType.DMA,
      ],
  )
  def k(emb_hbm, idx_hbm, out_hbm, idx_v, rows_v, sem):
    wid = lax.axis_index("s") * NC + lax.axis_index("c")
    base = wid * bpw
    pltpu.sync_copy(idx_hbm.at[pl.ds(base, bpw)], idx_v)
    pltpu.async_copy(emb_hbm.at[idx_v], rows_v, sem).wait()
    pltpu.sync_copy(rows_v, out_hbm.at[pl.ds(base, bpw)])

  return k(emb, idx_pad)


def _tc_project(x, basis, bn):
  N, Hin = x.shape
  nb, _, O = basis.shape
  assert N % bn == 0

  def body(x_ref, w_ref, y_ref):
    y_ref[...] = jnp.dot(x_ref[...], w_ref[0],
                         preferred_element_type=jnp.float32
                         ).astype(jnp.bfloat16)

  return pl.pallas_call(
      body,
      grid=(N // bn, nb),
      in_specs=[
          pl.BlockSpec((bn, Hin), lambda n, b: (n, 0)),
          pl.BlockSpec((1, Hin, O), lambda n, b: (b, 0, 0)),
      ],
      out_specs=pl.BlockSpec((bn, O), lambda n, b: (n, b)),
      out_shape=jax.ShapeDtypeStruct((N, nb * O), jnp.bfloat16),
  )(x, basis)


def _tc_combine(hp, bias, relu, bn):
  _, N, O = hp.shape
  bn = N
  assert N % bn == 0

  def body(h_ref, b_ref, o_ref):
    v = h_ref[0] + h_ref[1] + b_ref[...]
    if relu:
      v = jnp.maximum(v, 0.0)
    o_ref[...] = v

  return pl.pallas_call(
      body,
      grid=(N // bn,),
      in_specs=[
          pl.BlockSpec((2, bn, O), lambda n: (0, n, 0)),
          pl.BlockSpec((1, O), lambda n: (0, 0)),
      ],
      out_specs=pl.BlockSpec((bn, O), lambda n: (n, 0)),
      out_shape=jax.ShapeDtypeStruct((N, O), jnp.float32),
  )(hp, bias.reshape(1, O))


def _sc_edge_pass(y, src, dst, etype, norm, coeff_flat, zeros_init, ndst_pad, K):
  assert K % LANES == 0
  E = src.shape[0]
  D = y.shape[1]
  ep = (-E) % (NW * K)
  if ep:
    src = jnp.concatenate([src, jnp.zeros((ep,), src.dtype)])
    dst = jnp.concatenate([dst, jnp.zeros((ep,), dst.dtype)])
    etype = jnp.concatenate([etype, jnp.zeros((ep,), etype.dtype)])
    norm = jnp.concatenate([norm, jnp.zeros((ep,), norm.dtype)])
    E += ep
  epw = E // NW
  assert epw % K == 0
  nblk = epw // K
  rpt = ndst_pad // NS
  assert rpt * NS == ndst_pad and rpt % 8 == 0
  mesh = plsc.VectorSubcoreMesh(core_axis_name="c", subcore_axis_name="s")

  @functools.partial(
      pl.kernel,
      out_type=jax.ShapeDtypeStruct((NC * ndst_pad, H), jnp.float32),
      mesh=mesh,
      scratch_types=[
          pltpu.VMEM((epw,), jnp.int32),
          pltpu.VMEM((epw,), jnp.int32),
          pltpu.VMEM((epw,), jnp.float32),
          pltpu.VMEM((K,), jnp.int32),
          pltpu.VMEM((K,), jnp.int32),
          pltpu.VMEM((K,), jnp.int32),
          pltpu.VMEM((K, D), jnp.int32),
          pltpu.VMEM((K, D), jnp.int32),
          pltpu.VMEM((K, H), jnp.float32),
          pltpu.VMEM((coeff_flat.shape[0],), jnp.float32),
          pltpu.VMEM_SHARED((ndst_pad, H), jnp.float32),
          pltpu.SemaphoreType.DMA,
          pltpu.SemaphoreType.DMA,
          pltpu.SemaphoreType.DMA,
      ],
      compiler_params=pltpu.CompilerParams(needs_layout_passes=False),
  )
  def k(y_hbm, src_hbm, dst_hbm, et_hbm, nm_hbm, cf_hbm, z_hbm, out_hbm,
        src_v, et_v, nm_v, dst_v, sidx_a, sidx_b, rows_a, rows_b, m_v, cf_v,
        h_sh, sem_a, sem_b, sem_d):
    cid = lax.axis_index("c")
    sid = lax.axis_index("s")
    wid = cid * NS + sid

    pltpu.sync_copy(z_hbm, h_sh.at[pl.ds(sid * rpt, rpt)])
    pltpu.sync_copy(cf_hbm, cf_v)
    pltpu.sync_copy(src_hbm.at[pl.ds(wid * epw, epw)], src_v)
    pltpu.sync_copy(et_hbm.at[pl.ds(wid * epw, epw)], et_v)
    pltpu.sync_copy(nm_hbm.at[pl.ds(wid * epw, epw)], nm_v)
    plsc.subcore_barrier()

    def compute_block(i, rows_v):
      pltpu.async_copy(dst_hbm.at[pl.ds(wid * epw + i * K, K)], dst_v, sem_d)

      def edge_body(e, carry2):
        fvec = jnp.full((LANES,), 0, jnp.int32) + (i * K + e)
        et_b = plsc.load_gather(et_v, [fvec])
        nm_b = plsc.load_gather(nm_v, [fvec])
        acc = [jnp.zeros((LANES,), jnp.float32) for _ in range(H // LANES)]
        for b in range(NB):
          cb = plsc.load_gather(cf_v, [et_b * NB + b])
          for j in range(H // 32):
            w32 = rows_v[e, pl.ds(b * (H // 2) + j * LANES, LANES)]
            seg = plsc.bitcast(w32, jnp.bfloat16)
            lo, hi = plsc.unpack(seg, format=plsc.PackFormat.INTERLEAVED,
                                 preferred_element_type=jnp.float32)
            acc[2 * j] = acc[2 * j] + cb * lo
            acc[2 * j + 1] = acc[2 * j + 1] + cb * hi
        for j in range(H // LANES):
          m_v[e, pl.ds(j * LANES, LANES)] = acc[j] * nm_b
        return carry2

      lax.fori_loop(0, K, edge_body, 0)
      pltpu.make_async_copy(dst_hbm.at[pl.ds(0, K)], dst_v, sem_d).wait()
      pltpu.sync_copy(m_v, h_sh.at[dst_v], add=True)

    def gather_rows(i, sidx_ref, rows_ref, sem):
      for j in range(K // LANES):
        sidx_ref[pl.ds(j * LANES, LANES)] = src_v[pl.ds(i * K + j * LANES,
                                                        LANES)]
      pltpu.async_copy(y_hbm.at[sidx_ref], rows_ref, sem)

    def wait_rows(sidx_ref, rows_ref, sem):
      pltpu.make_async_copy(y_hbm.at[pl.ds(0, K)], rows_ref, sem).wait()

    gather_rows(0, sidx_a, rows_a, sem_a)

    def pair_body(i2, carry):
      b0 = 2 * i2
      wait_rows(sidx_a, rows_a, sem_a)
      gather_rows(b0 + 1, sidx_b, rows_b, sem_b)
      compute_block(b0, rows_a)
      wait_rows(sidx_b, rows_b, sem_b)
      gather_rows(jnp.minimum(b0 + 2, nblk - 1), sidx_a, rows_a, sem_a)
      compute_block(b0 + 1, rows_b)
      return carry

    lax.fori_loop(0, nblk // 2, pair_body, 0)
    wait_rows(sidx_a, rows_a, sem_a)
    if nblk % 2 == 1:
      compute_block(nblk - 1, rows_a)

    plsc.subcore_barrier()
    pltpu.sync_copy(h_sh.at[pl.ds(sid * rpt, rpt)],
                    out_hbm.at[pl.ds(cid * ndst_pad + sid * rpt, rpt)])

  out = k(y, src, dst, etype, norm, coeff_flat, zeros_init)
  return out.reshape(NC, ndst_pad, H)


def _layer(x_in, src, dst, etype, norm, basis, coeff, bias, num_dst, relu,
           zeros_init, ndst_pad, K, bn_proj, bn_comb):
  y = _tc_project(x_in, basis[:, :, _PERM], bn_proj)
  n_, d_ = y.shape
  y = lax.bitcast_convert_type(y.reshape(n_, d_ // 2, 2), jnp.int32)
  hp = _sc_edge_pass(y, src, dst, etype, norm.reshape(-1),
                     coeff.reshape(-1), zeros_init, ndst_pad, K)
  hp = hp[:, :num_dst, :]
  return _tc_combine(hp, bias, relu, bn_comb)


def kernel(node_ids, src1, dst1, etype1, norm1, src2, dst2, etype2, norm2,
           emb, basis1, coeff1, bias1, basis2, coeff2, bias2):
  n1 = node_ids.shape[0]
  pad = (-n1) % (8 * NW)
  idx_pad = jnp.concatenate([node_ids.astype(jnp.int32),
                             jnp.zeros((pad,), jnp.int32)])
  x = _sc_gather_rows(emb, idx_pad)[:n1]

  z1 = jnp.zeros((5120 // NS, H), jnp.float32)
  z2 = jnp.zeros((2560 // NS, H), jnp.float32)

  h = _layer(x, src1, dst1, etype1, norm1, basis1, coeff1, bias1,
             num_dst=5000, relu=True, zeros_init=z1, ndst_pad=5120,
             K=32, bn_proj=1000, bn_comb=500)
  h = _layer(h, src2, dst2, etype2, norm2, basis2, coeff2, bias2,
             num_dst=2500, relu=False, zeros_init=z2, ndst_pad=2560,
             K=32, bn_proj=1000, bn_comb=500)
  return h


# speedup vs baseline: 1.1061x; 1.1061x over previous
import functools

import jax
import jax.numpy as jnp
import numpy as np
from jax import lax
from jax.experimental import pallas as pl
from jax.experimental.pallas import tpu as pltpu
from jax.experimental.pallas import tpu_sc as plsc

NC = 2
NS = 16
NW = NC * NS
LANES = 16
NB = 8
H = 128

_PERM = np.empty((H,), np.int32)
for _j in range(H // 32):
  for _t in range(32):
    _PERM[_j * 32 + _t] = _j * 32 + (_t // 2 if _t % 2 == 0 else 16 + _t // 2)


def _sc_gather_rows(emb, idx_pad):
  B = idx_pad.shape[0]
  D = emb.shape[1]
  assert B % (8 * NW) == 0
  bpw = B // NW
  mesh = plsc.VectorSubcoreMesh(core_axis_name="c", subcore_axis_name="s")

  @functools.partial(
      pl.kernel,
      out_type=jax.ShapeDtypeStruct((B, D), jnp.float32),
      mesh=mesh,
      scratch_types=[
          pltpu.VMEM((bpw,), jnp.int32),
          pltpu.VMEM((bpw, D), jnp.float32),
          pltpu.SemaphoreType.DMA,
      ],
  )
  def k(emb_hbm, idx_hbm, out_hbm, idx_v, rows_v, sem):
    wid = lax.axis_index("s") * NC + lax.axis_index("c")
    base = wid * bpw
    pltpu.sync_copy(idx_hbm.at[pl.ds(base, bpw)], idx_v)
    pltpu.async_copy(emb_hbm.at[idx_v], rows_v, sem).wait()
    pltpu.sync_copy(rows_v, out_hbm.at[pl.ds(base, bpw)])

  return k(emb, idx_pad)


def _tc_project(x, basis, bn):
  N, Hin = x.shape
  nb, _, O = basis.shape
  assert N % bn == 0

  def body(x_ref, w_ref, y_ref):
    y_ref[...] = jnp.dot(x_ref[...], w_ref[0],
                         preferred_element_type=jnp.float32)

  return pl.pallas_call(
      body,
      grid=(N // bn, nb),
      in_specs=[
          pl.BlockSpec((bn, Hin), lambda n, b: (n, 0)),
          pl.BlockSpec((1, Hin, O), lambda n, b: (b, 0, 0)),
      ],
      out_specs=pl.BlockSpec((bn, O), lambda n, b: (n, b)),
      out_shape=jax.ShapeDtypeStruct((N, nb * O), jnp.float32),
  )(x, basis)


def _tc_combine(hp, bias, relu, bn):
  _, N, O = hp.shape
  bn = N
  assert N % bn == 0

  def body(h_ref, b_ref, o_ref):
    v = h_ref[0] + h_ref[1] + b_ref[...]
    if relu:
      v = jnp.maximum(v, 0.0)
    o_ref[...] = v

  return pl.pallas_call(
      body,
      grid=(N // bn,),
      in_specs=[
          pl.BlockSpec((2, bn, O), lambda n: (0, n, 0)),
          pl.BlockSpec((1, O), lambda n: (0, 0)),
      ],
      out_specs=pl.BlockSpec((bn, O), lambda n: (n, 0)),
      out_shape=jax.ShapeDtypeStruct((N, O), jnp.float32),
  )(hp, bias.reshape(1, O))


def _sc_edge_pass(y, src, dst, etype, norm, coeff_flat, zeros_init, ndst_pad, K):
  assert K % LANES == 0
  E = src.shape[0]
  D = y.shape[1]
  ep = (-E) % (NW * K)
  if ep:
    src = jnp.concatenate([src, jnp.zeros((ep,), src.dtype)])
    dst = jnp.concatenate([dst, jnp.zeros((ep,), dst.dtype)])
    etype = jnp.concatenate([etype, jnp.zeros((ep,), etype.dtype)])
    norm = jnp.concatenate([norm, jnp.zeros((ep,), norm.dtype)])
    E += ep
  epw = E // NW
  assert epw % K == 0
  nblk = epw // K
  rpt = ndst_pad // NS
  assert rpt * NS == ndst_pad and rpt % 8 == 0
  mesh = plsc.VectorSubcoreMesh(core_axis_name="c", subcore_axis_name="s")

  @functools.partial(
      pl.kernel,
      out_type=jax.ShapeDtypeStruct((NC * ndst_pad, H), jnp.float32),
      mesh=mesh,
      scratch_types=[
          pltpu.VMEM((epw,), jnp.int32),
          pltpu.VMEM((epw,), jnp.int32),
          pltpu.VMEM((epw,), jnp.float32),
          pltpu.VMEM((K,), jnp.int32),
          pltpu.VMEM((K,), jnp.int32),
          pltpu.VMEM((K,), jnp.int32),
          pltpu.VMEM((K,), jnp.int32),
          pltpu.VMEM((K, D), jnp.float32),
          pltpu.VMEM((K, D), jnp.float32),
          pltpu.VMEM((K, H), jnp.float32),
          pltpu.VMEM((K, H), jnp.float32),
          pltpu.VMEM((coeff_flat.shape[0],), jnp.float32),
          pltpu.VMEM_SHARED((ndst_pad, H), jnp.float32),
          pltpu.SemaphoreType.DMA,
          pltpu.SemaphoreType.DMA,
          pltpu.SemaphoreType.DMA,
          pltpu.SemaphoreType.DMA,
          pltpu.SemaphoreType.DMA,
          pltpu.SemaphoreType.DMA,
      ],
      compiler_params=pltpu.CompilerParams(needs_layout_passes=False),
  )
  def k(y_hbm, src_hbm, dst_hbm, et_hbm, nm_hbm, cf_hbm, z_hbm, out_hbm,
        src_v, et_v, nm_v, dst_a, dst_b, sidx_a, sidx_b, rows_a, rows_b,
        m_a, m_b, cf_v, h_sh, sem_a, sem_b, sem_da, sem_db, sem_ma, sem_mb):
    cid = lax.axis_index("c")
    sid = lax.axis_index("s")
    wid = cid * NS + sid

    pltpu.sync_copy(z_hbm, h_sh.at[pl.ds(sid * rpt, rpt)])
    pltpu.sync_copy(cf_hbm, cf_v)
    pltpu.sync_copy(src_hbm.at[pl.ds(wid * epw, epw)], src_v)
    pltpu.sync_copy(et_hbm.at[pl.ds(wid * epw, epw)], et_v)
    pltpu.sync_copy(nm_hbm.at[pl.ds(wid * epw, epw)], nm_v)
    plsc.subcore_barrier()

    def compute_block(i, rows_v, m_v, dst_v, sem_d, sem_m, do_wait):
      @pl.when(do_wait)
      def _():
        pltpu.make_async_copy(m_v, h_sh.at[dst_v], sem_m).wait()

      pltpu.async_copy(dst_hbm.at[pl.ds(wid * epw + i * K, K)], dst_v, sem_d)

      def edge_body(e, carry2):
        fvec = jnp.full((LANES,), 0, jnp.int32) + (i * K + e)
        et_b = plsc.load_gather(et_v, [fvec])
        nm_b = plsc.load_gather(nm_v, [fvec])
        acc = [jnp.zeros((LANES,), jnp.float32) for _ in range(H // LANES)]
        for b in range(NB):
          cb = plsc.load_gather(cf_v, [et_b * NB + b])
          for j in range(H // LANES):
            seg = rows_v[e, pl.ds(b * H + j * LANES, LANES)]
            acc[j] = acc[j] + cb * seg
        for j in range(H // LANES):
          m_v[e, pl.ds(j * LANES, LANES)] = acc[j] * nm_b
        return carry2

      lax.fori_loop(0, K, edge_body, 0)
      pltpu.make_async_copy(dst_hbm.at[pl.ds(0, K)], dst_v, sem_d).wait()
      pltpu.async_copy(m_v, h_sh.at[dst_v], sem_m, add=True)

    def gather_rows(i, sidx_ref, rows_ref, sem):
      for j in range(K // LANES):
        sidx_ref[pl.ds(j * LANES, LANES)] = src_v[pl.ds(i * K + j * LANES,
                                                        LANES)]
      pltpu.async_copy(y_hbm.at[sidx_ref], rows_ref, sem)

    def wait_rows(sidx_ref, rows_ref, sem):
      pltpu.make_async_copy(y_hbm.at[pl.ds(0, K)], rows_ref, sem).wait()

    gather_rows(0, sidx_a, rows_a, sem_a)

    def pair_body(i2, carry):
      b0 = 2 * i2
      wait_rows(sidx_a, rows_a, sem_a)
      gather_rows(b0 + 1, sidx_b, rows_b, sem_b)
      compute_block(b0, rows_a, m_a, dst_a, sem_da, sem_ma, i2 > 0)
      wait_rows(sidx_b, rows_b, sem_b)
      gather_rows(jnp.minimum(b0 + 2, nblk - 1), sidx_a, rows_a, sem_a)
      compute_block(b0 + 1, rows_b, m_b, dst_b, sem_db, sem_mb, i2 > 0)
      return carry

    npairs = nblk // 2
    lax.fori_loop(0, npairs, pair_body, 0)
    wait_rows(sidx_a, rows_a, sem_a)
    if nblk % 2 == 1:
      compute_block(nblk - 1, rows_a, m_a, dst_a, sem_da, sem_ma, npairs > 0)
    pltpu.make_async_copy(m_a, h_sh.at[dst_a], sem_ma).wait()
    pltpu.make_async_copy(m_b, h_sh.at[dst_b], sem_mb).wait()

    plsc.subcore_barrier()
    pltpu.sync_copy(h_sh.at[pl.ds(sid * rpt, rpt)],
                    out_hbm.at[pl.ds(cid * ndst_pad + sid * rpt, rpt)])

  out = k(y, src, dst, etype, norm, coeff_flat, zeros_init)
  return out.reshape(NC, ndst_pad, H)


def _layer(x_in, src, dst, etype, norm, basis, coeff, bias, num_dst, relu,
           zeros_init, ndst_pad, K, bn_proj, bn_comb):
  y = _tc_project(x_in, basis, bn_proj)
  hp = _sc_edge_pass(y, src, dst, etype, norm.reshape(-1),
                     coeff.reshape(-1), zeros_init, ndst_pad, K)
  hp = hp[:, :num_dst, :]
  return _tc_combine(hp, bias, relu, bn_comb)


def kernel(node_ids, src1, dst1, etype1, norm1, src2, dst2, etype2, norm2,
           emb, basis1, coeff1, bias1, basis2, coeff2, bias2):
  n1 = node_ids.shape[0]
  pad = (-n1) % (8 * NW)
  idx_pad = jnp.concatenate([node_ids.astype(jnp.int32),
                             jnp.zeros((pad,), jnp.int32)])
  x = _sc_gather_rows(emb, idx_pad)[:n1]

  z1 = jnp.zeros((5120 // NS, H), jnp.float32)
  z2 = jnp.zeros((2560 // NS, H), jnp.float32)

  h = _layer(x, src1, dst1, etype1, norm1, basis1, coeff1, bias1,
             num_dst=5000, relu=True, zeros_init=z1, ndst_pad=5120,
             K=16, bn_proj=1000, bn_comb=500)
  h = _layer(h, src2, dst2, etype2, norm2, basis2, coeff2, bias2,
             num_dst=2500, relu=False, zeros_init=z2, ndst_pad=2560,
             K=16, bn_proj=1000, bn_comb=500)
  return h


# speedup vs baseline: 1.3422x; 1.2135x over previous
import functools

import jax
import jax.numpy as jnp
import numpy as np
from jax import lax
from jax.experimental import pallas as pl
from jax.experimental.pallas import tpu as pltpu
from jax.experimental.pallas import tpu_sc as plsc

NC = 2
NS = 16
NW = NC * NS
LANES = 16
NB = 8
H = 128

_PERM = np.empty((H,), np.int32)
for _j in range(H // 32):
  for _t in range(32):
    _PERM[_j * 32 + _t] = _j * 32 + (_t // 2 if _t % 2 == 0 else 16 + _t // 2)


def _sc_gather_rows(emb, idx_pad):
  B = idx_pad.shape[0]
  D = emb.shape[1]
  assert B % (8 * NW) == 0
  bpw = B // NW
  mesh = plsc.VectorSubcoreMesh(core_axis_name="c", subcore_axis_name="s")

  @functools.partial(
      pl.kernel,
      out_type=jax.ShapeDtypeStruct((B, D), jnp.float32),
      mesh=mesh,
      scratch_types=[
          pltpu.VMEM((bpw,), jnp.int32),
          pltpu.VMEM((bpw, D), jnp.float32),
          pltpu.SemaphoreType.DMA,
      ],
  )
  def k(emb_hbm, idx_hbm, out_hbm, idx_v, rows_v, sem):
    wid = lax.axis_index("s") * NC + lax.axis_index("c")
    base = wid * bpw
    pltpu.sync_copy(idx_hbm.at[pl.ds(base, bpw)], idx_v)
    pltpu.async_copy(emb_hbm.at[idx_v], rows_v, sem).wait()
    pltpu.sync_copy(rows_v, out_hbm.at[pl.ds(base, bpw)])

  return k(emb, idx_pad)


def _tc_project(x, basis, bn):
  N, Hin = x.shape
  nb, _, O = basis.shape
  assert N % bn == 0

  def body(x_ref, w_ref, y_ref):
    y_ref[...] = jnp.dot(x_ref[...], w_ref[0],
                         preferred_element_type=jnp.float32)

  return pl.pallas_call(
      body,
      grid=(N // bn, nb),
      in_specs=[
          pl.BlockSpec((bn, Hin), lambda n, b: (n, 0)),
          pl.BlockSpec((1, Hin, O), lambda n, b: (b, 0, 0)),
      ],
      out_specs=pl.BlockSpec((bn, O), lambda n, b: (n, b)),
      out_shape=jax.ShapeDtypeStruct((N, nb * O), jnp.float32),
  )(x, basis)


def _tc_combine(hp, bias, relu, bn):
  _, N, O = hp.shape
  bn = N
  assert N % bn == 0

  def body(h_ref, b_ref, o_ref):
    v = h_ref[0] + h_ref[1] + b_ref[...]
    if relu:
      v = jnp.maximum(v, 0.0)
    o_ref[...] = v

  return pl.pallas_call(
      body,
      grid=(N // bn,),
      in_specs=[
          pl.BlockSpec((2, bn, O), lambda n: (0, n, 0)),
          pl.BlockSpec((1, O), lambda n: (0, 0)),
      ],
      out_specs=pl.BlockSpec((bn, O), lambda n: (n, 0)),
      out_shape=jax.ShapeDtypeStruct((N, O), jnp.float32),
  )(hp, bias.reshape(1, O))


def _sc_edge_pass(y, src, dst, etype, norm, coeff_flat, zeros_init, ndst_pad, K):
  assert K % LANES == 0
  E = src.shape[0]
  D = y.shape[1]
  ep = (-E) % (NW * K)
  if ep:
    src = jnp.concatenate([src, jnp.zeros((ep,), src.dtype)])
    dst = jnp.concatenate([dst, jnp.zeros((ep,), dst.dtype)])
    etype = jnp.concatenate([etype, jnp.zeros((ep,), etype.dtype)])
    norm = jnp.concatenate([norm, jnp.zeros((ep,), norm.dtype)])
    E += ep
  epw = E // NW
  assert epw % K == 0
  nblk = epw // K
  den3 = jnp.stack([etype.reshape(NW, nblk, K),
                    lax.bitcast_convert_type(norm, jnp.int32
                                             ).reshape(NW, nblk, K)], axis=2)
  rpt = ndst_pad // NS
  assert rpt * NS == ndst_pad and rpt % 8 == 0
  mesh = plsc.VectorSubcoreMesh(core_axis_name="c", subcore_axis_name="s")

  @functools.partial(
      pl.kernel,
      out_type=jax.ShapeDtypeStruct((NC * ndst_pad, H), jnp.float32),
      mesh=mesh,
      scratch_types=[
          pltpu.VMEM((epw,), jnp.int32),
          pltpu.VMEM((2, K), jnp.int32),
          pltpu.VMEM((2, K), jnp.int32),
          pltpu.VMEM((K,), jnp.int32),
          pltpu.VMEM((K,), jnp.int32),
          pltpu.VMEM((K,), jnp.int32),
          pltpu.VMEM((K,), jnp.int32),
          pltpu.VMEM((K, D), jnp.float32),
          pltpu.VMEM((K, D), jnp.float32),
          pltpu.VMEM((K, H), jnp.float32),
          pltpu.VMEM((K, H), jnp.float32),
          pltpu.VMEM((coeff_flat.shape[0],), jnp.float32),
          pltpu.VMEM_SHARED((ndst_pad, H), jnp.float32),
          pltpu.SemaphoreType.DMA,
          pltpu.SemaphoreType.DMA,
          pltpu.SemaphoreType.DMA,
          pltpu.SemaphoreType.DMA,
          pltpu.SemaphoreType.DMA,
          pltpu.SemaphoreType.DMA,
          pltpu.SemaphoreType.DMA,
          pltpu.SemaphoreType.DMA,
      ],
      compiler_params=pltpu.CompilerParams(needs_layout_passes=False),
  )
  def k(y_hbm, src_hbm, dst_hbm, den_hbm, cf_hbm, z_hbm, out_hbm,
        src_v, den_a, den_b, dst_a, dst_b, sidx_a, sidx_b, rows_a, rows_b,
        m_a, m_b, cf_v, h_sh, sem_a, sem_b, sem_da, sem_db, sem_ma, sem_mb,
        sem_ea, sem_eb):
    cid = lax.axis_index("c")
    sid = lax.axis_index("s")
    wid = cid * NS + sid

    pltpu.sync_copy(z_hbm, h_sh.at[pl.ds(sid * rpt, rpt)])
    pltpu.sync_copy(cf_hbm, cf_v)
    pltpu.sync_copy(src_hbm.at[pl.ds(wid * epw, epw)], src_v)
    pltpu.sync_copy(den_hbm.at[wid, 0], den_a)
    plsc.subcore_barrier()

    def compute_block(i, rows_v, m_v, dst_v, den_cur, sem_ecur, den_nxt,
                      sem_enxt, sem_d, sem_m, do_wait):
      @pl.when(do_wait)
      def _():
        pltpu.make_async_copy(m_v, h_sh.at[dst_v], sem_m).wait()

      pltpu.async_copy(dst_hbm.at[pl.ds(wid * epw + i * K, K)], dst_v, sem_d)
      pltpu.async_copy(den_hbm.at[wid, jnp.minimum(i + 1, nblk - 1)],
                       den_nxt, sem_enxt)
      @pl.when(i > 0)
      def _():
        pltpu.make_async_copy(den_hbm.at[wid, 0], den_cur, sem_ecur).wait()

      def edge_body(e, carry2):
        evec = jnp.full((LANES,), 0, jnp.int32) + e
        et_b = plsc.load_gather(den_cur, [evec * 0, evec])
        nm_b = plsc.bitcast(
            plsc.load_gather(den_cur, [evec * 0 + 1, evec]), jnp.float32)
        acc = [jnp.zeros((LANES,), jnp.float32) for _ in range(H // LANES)]
        for b in range(NB):
          cb = plsc.load_gather(cf_v, [et_b * NB + b])
          for j in range(H // LANES):
            seg = rows_v[e, pl.ds(b * H + j * LANES, LANES)]
            acc[j] = acc[j] + cb * seg
        for j in range(H // LANES):
          m_v[e, pl.ds(j * LANES, LANES)] = acc[j] * nm_b
        return carry2

      lax.fori_loop(0, K, edge_body, 0)
      pltpu.make_async_copy(dst_hbm.at[pl.ds(0, K)], dst_v, sem_d).wait()
      pltpu.async_copy(m_v, h_sh.at[dst_v], sem_m, add=True)

    def gather_rows(i, sidx_ref, rows_ref, sem):
      for j in range(K // LANES):
        sidx_ref[pl.ds(j * LANES, LANES)] = src_v[pl.ds(i * K + j * LANES,
                                                        LANES)]
      pltpu.async_copy(y_hbm.at[sidx_ref], rows_ref, sem)

    def wait_rows(sidx_ref, rows_ref, sem):
      pltpu.make_async_copy(y_hbm.at[pl.ds(0, K)], rows_ref, sem).wait()

    gather_rows(0, sidx_a, rows_a, sem_a)

    def pair_body(i2, carry):
      b0 = 2 * i2
      wait_rows(sidx_a, rows_a, sem_a)
      gather_rows(b0 + 1, sidx_b, rows_b, sem_b)
      compute_block(b0, rows_a, m_a, dst_a, den_a, sem_ea, den_b, sem_eb,
                    sem_da, sem_ma, i2 > 0)
      wait_rows(sidx_b, rows_b, sem_b)
      gather_rows(jnp.minimum(b0 + 2, nblk - 1), sidx_a, rows_a, sem_a)
      compute_block(b0 + 1, rows_b, m_b, dst_b, den_b, sem_eb, den_a, sem_ea,
                    sem_db, sem_mb, i2 > 0)
      return carry

    npairs = nblk // 2
    lax.fori_loop(0, npairs, pair_body, 0)
    wait_rows(sidx_a, rows_a, sem_a)
    if nblk % 2 == 1:
      compute_block(nblk - 1, rows_a, m_a, dst_a, den_a, sem_ea, den_b,
                    sem_eb, sem_da, sem_ma, npairs > 0)
    pltpu.make_async_copy(m_a, h_sh.at[dst_a], sem_ma).wait()
    pltpu.make_async_copy(m_b, h_sh.at[dst_b], sem_mb).wait()
    if nblk % 2 == 1:
      pltpu.make_async_copy(den_hbm.at[wid, 0], den_b, sem_eb).wait()
    else:
      pltpu.make_async_copy(den_hbm.at[wid, 0], den_a, sem_ea).wait()

    plsc.subcore_barrier()
    pltpu.sync_copy(h_sh.at[pl.ds(sid * rpt, rpt)],
                    out_hbm.at[pl.ds(cid * ndst_pad + sid * rpt, rpt)])

  out = k(y, src, dst, den3, coeff_flat, zeros_init)
  return out.reshape(NC, ndst_pad, H)


def _layer(x_in, src, dst, etype, norm, basis, coeff, bias, num_dst, relu,
           zeros_init, ndst_pad, K, bn_proj, bn_comb):
  y = _tc_project(x_in, basis, bn_proj)
  hp = _sc_edge_pass(y, src, dst, etype, norm.reshape(-1),
                     coeff.reshape(-1), zeros_init, ndst_pad, K)
  hp = hp[:, :num_dst, :]
  return _tc_combine(hp, bias, relu, bn_comb)


def kernel(node_ids, src1, dst1, etype1, norm1, src2, dst2, etype2, norm2,
           emb, basis1, coeff1, bias1, basis2, coeff2, bias2):
  n1 = node_ids.shape[0]
  pad = (-n1) % (8 * NW)
  idx_pad = jnp.concatenate([node_ids.astype(jnp.int32),
                             jnp.zeros((pad,), jnp.int32)])
  x = _sc_gather_rows(emb, idx_pad)[:n1]

  z1 = jnp.zeros((5120 // NS, H), jnp.float32)
  z2 = jnp.zeros((2560 // NS, H), jnp.float32)

  h = _layer(x, src1, dst1, etype1, norm1, basis1, coeff1, bias1,
             num_dst=5000, relu=True, zeros_init=z1, ndst_pad=5120,
             K=32, bn_proj=1000, bn_comb=500)
  h = _layer(h, src2, dst2, etype2, norm2, basis2, coeff2, bias2,
             num_dst=2500, relu=False, zeros_init=z2, ndst_pad=2560,
             K=32, bn_proj=1000, bn_comb=500)
  return h


# speedup vs baseline: 1.3501x; 1.0059x over previous
import functools

import jax
import jax.numpy as jnp
from jax import lax
from jax.experimental import pallas as pl
from jax.experimental.pallas import tpu as pltpu
from jax.experimental.pallas import tpu_sc as plsc

NC = 2
NS = 16
NW = NC * NS
LANES = 16
NB = 8
H = 128


def _sc_gather_rows(emb, idx_pad):
  B = idx_pad.shape[0]
  D = emb.shape[1]
  assert B % (8 * NW) == 0
  bpw = B // NW
  mesh = plsc.VectorSubcoreMesh(core_axis_name="c", subcore_axis_name="s")

  @functools.partial(
      pl.kernel,
      out_type=jax.ShapeDtypeStruct((B, D), jnp.float32),
      mesh=mesh,
      scratch_types=[
          pltpu.VMEM((bpw,), jnp.int32),
          pltpu.VMEM((bpw, D), jnp.float32),
          pltpu.SemaphoreType.DMA,
      ],
  )
  def k(emb_hbm, idx_hbm, out_hbm, idx_v, rows_v, sem):
    wid = lax.axis_index("s") * NC + lax.axis_index("c")
    base = wid * bpw
    pltpu.sync_copy(idx_hbm.at[pl.ds(base, bpw)], idx_v)
    pltpu.async_copy(emb_hbm.at[idx_v], rows_v, sem).wait()
    pltpu.sync_copy(rows_v, out_hbm.at[pl.ds(base, bpw)])

  return k(emb, idx_pad)


def _tc_project(x, basis, bn):
  N, Hin = x.shape
  nb, _, O = basis.shape
  assert N % bn == 0

  def body(x_ref, w_ref, y_ref):
    y_ref[...] = jnp.dot(x_ref[...], w_ref[0],
                         preferred_element_type=jnp.float32)

  return pl.pallas_call(
      body,
      grid=(N // bn, nb),
      in_specs=[
          pl.BlockSpec((bn, Hin), lambda n, b: (n, 0)),
          pl.BlockSpec((1, Hin, O), lambda n, b: (b, 0, 0)),
      ],
      out_specs=pl.BlockSpec((bn, O), lambda n, b: (n, b)),
      out_shape=jax.ShapeDtypeStruct((N, nb * O), jnp.float32),
  )(x, basis)


def _tc_combine_project(hp, bias, basis, bn):
  _, N, O = hp.shape
  nb = basis.shape[0]
  Oo = basis.shape[2]
  assert N % bn == 0

  def body(h_ref, b_ref, w_ref, y_ref):
    x2 = jnp.maximum(h_ref[0] + h_ref[1] + b_ref[...], 0.0)
    y_ref[...] = jnp.dot(x2, w_ref[0], preferred_element_type=jnp.float32)

  return pl.pallas_call(
      body,
      grid=(N // bn, nb),
      in_specs=[
          pl.BlockSpec((2, bn, O), lambda n, b: (0, n, 0)),
          pl.BlockSpec((1, O), lambda n, b: (0, 0)),
          pl.BlockSpec((1, O, Oo), lambda n, b: (b, 0, 0)),
      ],
      out_specs=pl.BlockSpec((bn, Oo), lambda n, b: (n, b)),
      out_shape=jax.ShapeDtypeStruct((N, nb * Oo), jnp.float32),
  )(hp, bias.reshape(1, O), basis)


def _tc_combine(hp, bias, relu, bn):
  _, N, O = hp.shape
  bn = N
  assert N % bn == 0

  def body(h_ref, b_ref, o_ref):
    v = h_ref[0] + h_ref[1] + b_ref[...]
    if relu:
      v = jnp.maximum(v, 0.0)
    o_ref[...] = v

  return pl.pallas_call(
      body,
      grid=(N // bn,),
      in_specs=[
          pl.BlockSpec((2, bn, O), lambda n: (0, n, 0)),
          pl.BlockSpec((1, O), lambda n: (0, 0)),
      ],
      out_specs=pl.BlockSpec((bn, O), lambda n: (n, 0)),
      out_shape=jax.ShapeDtypeStruct((N, O), jnp.float32),
  )(hp, bias.reshape(1, O))


def _sc_edge_pass(y, src, dst, etype, norm, coeff_flat, zeros_init, ndst_pad, K):
  assert K % LANES == 0
  E = src.shape[0]
  D = y.shape[1]
  ep = (-E) % (NW * K)
  if ep:
    src = jnp.concatenate([src, jnp.zeros((ep,), src.dtype)])
    dst = jnp.concatenate([dst, jnp.zeros((ep,), dst.dtype)])
    etype = jnp.concatenate([etype, jnp.zeros((ep,), etype.dtype)])
    norm = jnp.concatenate([norm, jnp.zeros((ep,), norm.dtype)])
    E += ep
  epw = E // NW
  assert epw % K == 0
  nblk = epw // K
  den3 = jnp.stack([etype.reshape(NW, nblk, K),
                    lax.bitcast_convert_type(norm, jnp.int32
                                             ).reshape(NW, nblk, K)], axis=2)
  rpt = ndst_pad // NS
  assert rpt * NS == ndst_pad and rpt % 8 == 0
  mesh = plsc.VectorSubcoreMesh(core_axis_name="c", subcore_axis_name="s")

  @functools.partial(
      pl.kernel,
      out_type=jax.ShapeDtypeStruct((NC * ndst_pad, H), jnp.float32),
      mesh=mesh,
      scratch_types=[
          pltpu.VMEM((epw,), jnp.int32),
          pltpu.VMEM((2, K), jnp.int32),
          pltpu.VMEM((2, K), jnp.int32),
          pltpu.VMEM((K,), jnp.int32),
          pltpu.VMEM((K,), jnp.int32),
          pltpu.VMEM((K,), jnp.int32),
          pltpu.VMEM((K,), jnp.int32),
          pltpu.VMEM((K, D), jnp.float32),
          pltpu.VMEM((K, D), jnp.float32),
          pltpu.VMEM((K, H), jnp.float32),
          pltpu.VMEM((K, H), jnp.float32),
          pltpu.VMEM((coeff_flat.shape[0],), jnp.float32),
          pltpu.VMEM_SHARED((ndst_pad, H), jnp.float32),
          pltpu.SemaphoreType.DMA,
          pltpu.SemaphoreType.DMA,
          pltpu.SemaphoreType.DMA,
          pltpu.SemaphoreType.DMA,
          pltpu.SemaphoreType.DMA,
          pltpu.SemaphoreType.DMA,
          pltpu.SemaphoreType.DMA,
          pltpu.SemaphoreType.DMA,
      ],
      compiler_params=pltpu.CompilerParams(needs_layout_passes=False),
  )
  def k(y_hbm, src_hbm, dst_hbm, den_hbm, cf_hbm, z_hbm, out_hbm,
        src_v, den_a, den_b, dst_a, dst_b, sidx_a, sidx_b, rows_a, rows_b,
        m_a, m_b, cf_v, h_sh, sem_a, sem_b, sem_da, sem_db, sem_ma, sem_mb,
        sem_ea, sem_eb):
    cid = lax.axis_index("c")
    sid = lax.axis_index("s")
    wid = cid * NS + sid

    pltpu.sync_copy(z_hbm, h_sh.at[pl.ds(sid * rpt, rpt)])
    pltpu.sync_copy(cf_hbm, cf_v)
    pltpu.sync_copy(src_hbm.at[pl.ds(wid * epw, epw)], src_v)
    pltpu.sync_copy(den_hbm.at[wid, 0], den_a)
    plsc.subcore_barrier()

    def compute_block(i, rows_v, m_v, dst_v, den_cur, sem_ecur, den_nxt,
                      sem_enxt, sem_d, sem_m, do_wait):
      @pl.when(do_wait)
      def _():
        pltpu.make_async_copy(m_v, h_sh.at[dst_v], sem_m).wait()

      pltpu.async_copy(dst_hbm.at[pl.ds(wid * epw + i * K, K)], dst_v, sem_d)
      pltpu.async_copy(den_hbm.at[wid, jnp.minimum(i + 1, nblk - 1)],
                       den_nxt, sem_enxt)
      @pl.when(i > 0)
      def _():
        pltpu.make_async_copy(den_hbm.at[wid, 0], den_cur, sem_ecur).wait()

      def edge_body(e, carry2):
        evec = jnp.full((LANES,), 0, jnp.int32) + e
        et_b = plsc.load_gather(den_cur, [evec * 0, evec])
        nm_b = plsc.bitcast(
            plsc.load_gather(den_cur, [evec * 0 + 1, evec]), jnp.float32)
        acc = [jnp.zeros((LANES,), jnp.float32) for _ in range(H // LANES)]
        for b in range(NB):
          cb = plsc.load_gather(cf_v, [et_b * NB + b])
          for j in range(H // LANES):
            seg = rows_v[e, pl.ds(b * H + j * LANES, LANES)]
            acc[j] = acc[j] + cb * seg
        for j in range(H // LANES):
          m_v[e, pl.ds(j * LANES, LANES)] = acc[j] * nm_b
        return carry2

      lax.fori_loop(0, K, edge_body, 0)
      pltpu.make_async_copy(dst_hbm.at[pl.ds(0, K)], dst_v, sem_d).wait()
      pltpu.async_copy(m_v, h_sh.at[dst_v], sem_m, add=True)

    def gather_rows(i, sidx_ref, rows_ref, sem):
      for j in range(K // LANES):
        sidx_ref[pl.ds(j * LANES, LANES)] = src_v[pl.ds(i * K + j * LANES,
                                                        LANES)]
      pltpu.async_copy(y_hbm.at[sidx_ref], rows_ref, sem)

    def wait_rows(sidx_ref, rows_ref, sem):
      pltpu.make_async_copy(y_hbm.at[pl.ds(0, K)], rows_ref, sem).wait()

    gather_rows(0, sidx_a, rows_a, sem_a)

    def pair_body(i2, carry):
      b0 = 2 * i2
      wait_rows(sidx_a, rows_a, sem_a)
      gather_rows(b0 + 1, sidx_b, rows_b, sem_b)
      compute_block(b0, rows_a, m_a, dst_a, den_a, sem_ea, den_b, sem_eb,
                    sem_da, sem_ma, i2 > 0)
      wait_rows(sidx_b, rows_b, sem_b)
      gather_rows(jnp.minimum(b0 + 2, nblk - 1), sidx_a, rows_a, sem_a)
      compute_block(b0 + 1, rows_b, m_b, dst_b, den_b, sem_eb, den_a, sem_ea,
                    sem_db, sem_mb, i2 > 0)
      return carry

    npairs = nblk // 2
    lax.fori_loop(0, npairs, pair_body, 0)
    wait_rows(sidx_a, rows_a, sem_a)
    if nblk % 2 == 1:
      compute_block(nblk - 1, rows_a, m_a, dst_a, den_a, sem_ea, den_b,
                    sem_eb, sem_da, sem_ma, npairs > 0)
    pltpu.make_async_copy(m_a, h_sh.at[dst_a], sem_ma).wait()
    pltpu.make_async_copy(m_b, h_sh.at[dst_b], sem_mb).wait()
    if nblk % 2 == 1:
      pltpu.make_async_copy(den_hbm.at[wid, 0], den_b, sem_eb).wait()
    else:
      pltpu.make_async_copy(den_hbm.at[wid, 0], den_a, sem_ea).wait()

    plsc.subcore_barrier()
    pltpu.sync_copy(h_sh.at[pl.ds(sid * rpt, rpt)],
                    out_hbm.at[pl.ds(cid * ndst_pad + sid * rpt, rpt)])

  out = k(y, src, dst, den3, coeff_flat, zeros_init)
  return out.reshape(NC, ndst_pad, H)


def kernel(node_ids, src1, dst1, etype1, norm1, src2, dst2, etype2, norm2,
           emb, basis1, coeff1, bias1, basis2, coeff2, bias2):
  n1 = node_ids.shape[0]
  pad = (-n1) % (8 * NW)
  idx_pad = jnp.concatenate([node_ids.astype(jnp.int32),
                             jnp.zeros((pad,), jnp.int32)])
  x = _sc_gather_rows(emb, idx_pad)[:n1]

  z1 = jnp.zeros((5120 // NS, H), jnp.float32)
  z2 = jnp.zeros((2560 // NS, H), jnp.float32)

  y1 = _tc_project(x, basis1, 1000)
  hp1 = _sc_edge_pass(y1, src1, dst1, etype1, norm1.reshape(-1),
                      coeff1.reshape(-1), z1, 5120, 32)[:, :5000]
  y2 = _tc_combine_project(hp1, bias1, basis2, 1000)
  hp2 = _sc_edge_pass(y2, src2, dst2, etype2, norm2.reshape(-1),
                      coeff2.reshape(-1), z2, 2560, 32)[:, :2500]
  return _tc_combine(hp2, bias2, relu=False, bn=500)
